# Initial kernel scaffold; baseline (speedup 1.0000x reference)
import functools

import jax
import jax.numpy as jnp
from jax import lax
from jax.experimental import pallas as pl
from jax.experimental.pallas import tpu as pltpu
from jax.experimental.pallas import tpu_sc as plsc

L = 16
K = 80
ROWW = 144


def _tc_pre_body(h_ref, wz_ref, wzi_ref, v_ref, z_ref, zi_ref, pq_ref):
    hb = h_ref[...]
    z_ref[...] = jnp.dot(hb, wz_ref[...], preferred_element_type=jnp.float32)
    zi_ref[...] = jnp.dot(hb, wzi_ref[...], preferred_element_type=jnp.float32)
    pq_ref[...] = jnp.dot(hb, v_ref[...], preferred_element_type=jnp.float32)


def _tc_pre(h, Wz, Wzi, V, block=1000):
    n, dd = h.shape
    hh = Wz.shape[1]
    return pl.pallas_call(
        _tc_pre_body,
        grid=(n // block,),
        in_specs=[
            pl.BlockSpec((block, dd), lambda i: (i, 0)),
            pl.BlockSpec((dd, hh), lambda i: (0, 0)),
            pl.BlockSpec((dd, hh), lambda i: (0, 0)),
            pl.BlockSpec((dd, 2), lambda i: (0, 0)),
        ],
        out_specs=[
            pl.BlockSpec((block, hh), lambda i: (i, 0)),
            pl.BlockSpec((block, hh), lambda i: (i, 0)),
            pl.BlockSpec((block, 2), lambda i: (i, 0)),
        ],
        out_shape=[
            jax.ShapeDtypeStruct((n, hh), jnp.float32),
            jax.ShapeDtypeStruct((n, hh), jnp.float32),
            jax.ShapeDtypeStruct((n, 2), jnp.float32),
        ],
    )(h, Wz, Wzi, V)


def _tc_post_body(s_ref, m_ref, zi_ref, o_ref):
    mv = m_ref[...]
    mm = jnp.max(mv)
    wv = jnp.exp(mv - mm)
    s = s_ref[0] * wv[0, 0] + s_ref[1] * wv[1, 0]
    num = s[:, :128]
    den = s[:, 128:129]
    zn = jnp.where(den > 0, num / den, 0.0)
    o_ref[...] = jnp.maximum(zi_ref[...] + zn, 0.0)


def _tc_post(S2, m2, zi, block=1000):
    n, hh = zi.shape
    return pl.pallas_call(
        _tc_post_body,
        grid=(n // block,),
        in_specs=[
            pl.BlockSpec((2, block, ROWW), lambda i: (0, i, 0)),
            pl.BlockSpec((2, L), lambda i: (0, 0)),
            pl.BlockSpec((block, hh), lambda i: (i, 0)),
        ],
        out_specs=pl.BlockSpec((block, hh), lambda i: (i, 0)),
        out_shape=jax.ShapeDtypeStruct((n, hh), jnp.float32),
    )(S2, m2, zi)


def _make_sc_edge(n, e, hh):
    info = plsc.get_sparse_core_info()
    nc, ns = info.num_cores, info.num_subcores
    nw = nc * ns
    ep = e // nw
    c2 = ep // K
    rows_per = n // ns
    zb = 125
    mesh = plsc.VectorSubcoreMesh(core_axis_name="c", subcore_axis_name="s")

    @functools.partial(
        pl.kernel,
        out_type=[
            jax.ShapeDtypeStruct((nc, n, ROWW), jnp.float32),
            jax.ShapeDtypeStruct((nc, L), jnp.float32),
        ],
        mesh=mesh,
        scratch_types=[
            pltpu.VMEM((c2, K), jnp.int32),
            pltpu.VMEM((c2, K), jnp.int32),
            pltpu.VMEM((ep,), jnp.float32),
            pltpu.VMEM((2 * n,), jnp.float32),
            pltpu.VMEM((ep,), jnp.float32),
            pltpu.VMEM((K, hh), jnp.float32),
            pltpu.VMEM((K, ROWW), jnp.float32),
            pltpu.VMEM((L,), jnp.float32),
            pltpu.VMEM((L,), jnp.float32),
            pltpu.VMEM((ns, L), jnp.float32),
            pltpu.VMEM((125, ROWW), jnp.float32),
            pltpu.VMEM_SHARED((n, ROWW), jnp.float32),
            pltpu.VMEM_SHARED((ns, L), jnp.float32),
            pltpu.SemaphoreType.DMA,
        ],
    )
    def sc_edge(z_hbm, pq_hbm, d_hbm, src_hbm, dst_hbm, c_hbm, s_out, m_out,
                src_v, dst_v, d_v, pq_v, e_v, zrows_v, srows_v, c_v,
                stage_v, allmax_v, zero_v, s_sh, max_sh, sem):
        cid = lax.axis_index("c")
        sid = lax.axis_index("s")
        wid = cid * ns + sid
        pltpu.sync_copy(src_hbm.at[pl.ds(wid * c2, c2)], src_v)
        pltpu.sync_copy(dst_hbm.at[pl.ds(wid * c2, c2)], dst_v)
        pltpu.sync_copy(d_hbm.at[pl.ds(wid * ep, ep)], d_v)
        pltpu.sync_copy(pq_hbm, pq_v)
        pltpu.sync_copy(c_hbm, c_v)
        cvec = c_v[...]

        def p1(i, mx):
            jc = i // (K // L)
            u = i % (K // L)
            sv = src_v[jc, pl.ds(u * L, L)]
            dv = dst_v[jc, pl.ds(u * L, L)]
            dd = d_v[pl.ds(i * L, L)]
            pg = plsc.load_gather(pq_v, [sv * 2])
            qg = plsc.load_gather(pq_v, [dv * 2 + 1])
            a = pg + qg + cvec * dd
            ev = jnp.maximum(a, 0.01 * a)
            e_v[pl.ds(i * L, L)] = ev
            return jnp.maximum(mx, ev)

        mx = lax.fori_loop(0, ep // L, p1,
                           jnp.full((L,), -jnp.inf, jnp.float32))

        def zfill(i, _):
            def zrow(j, _):
                zero_v[i, pl.ds(j * L, L)] = jnp.zeros((L,), jnp.float32)
                return 0
            lax.fori_loop(0, ROWW // L, zrow, 0)
            return 0
        lax.fori_loop(0, 125, zfill, 0)
        for t in range(rows_per // 125):
            pltpu.sync_copy(zero_v,
                            s_sh.at[pl.ds(sid * rows_per + t * 125, 125)])

        stage_v[...] = mx
        pltpu.sync_copy(stage_v, max_sh.at[sid])
        plsc.subcore_barrier()
        pltpu.sync_copy(max_sh, allmax_v)

        def rmax(i, acc):
            return jnp.maximum(acc, allmax_v[i])
        mxv = lax.fori_loop(0, ns, rmax,
                            jnp.full((L,), -jnp.inf, jnp.float32))
        m = jnp.max(mxv)

        ii = lax.iota(jnp.int32, L)
        ohv = jnp.where(ii == 0, 1.0, 0.0).astype(jnp.float32)

        def p2(jc, _):
            for u in range(K // L):
                sl = pl.ds(jc * K + u * L, L)
                e_v[sl] = jnp.exp(e_v[sl] - m)
            cp = pltpu.async_copy(z_hbm.at[src_v.at[jc]], zrows_v, sem)
            cp.wait()

            def prow(r, _):
                s = e_v[jc * K + r]
                sv16 = jnp.full((L,), s, jnp.float32)
                for cc in range(hh // L):
                    srows_v[r, pl.ds(cc * L, L)] = (
                        zrows_v[r, pl.ds(cc * L, L)] * sv16)
                srows_v[r, pl.ds(hh, L)] = sv16 * ohv
                return 0
            lax.fori_loop(0, K, prow, 0)
            pltpu.sync_copy(srows_v, s_sh.at[dst_v.at[jc]], add=True)
            return 0
        lax.fori_loop(0, c2, p2, 0)
        plsc.subcore_barrier()

        pltpu.sync_copy(s_sh.at[pl.ds(sid * rows_per, rows_per)],
                        s_out.at[cid, pl.ds(sid * rows_per, rows_per)])

        @pl.when(sid == 0)
        def _():
            stage_v[...] = jnp.full((L,), m, jnp.float32)
            pltpu.sync_copy(stage_v, m_out.at[cid])

    return sc_edge


def _layer(h, d1, src_r, dst_r, W0, W1, W2, Wa, sc_edge):
    hh = W1.shape[0]
    wa1 = Wa[0, :hh]
    wa2 = Wa[0, hh:2 * hh]
    c = W0[0, 0] * Wa[0, 2 * hh]
    Wz = W1.T
    Wzi = W2.T
    V = jnp.stack([W1.T @ wa1, W1.T @ wa2], axis=1)
    c16 = jnp.full((L,), c, jnp.float32)
    z, zi, pq = _tc_pre(h, Wz, Wzi, V)
    S2, m2 = sc_edge(z, pq.reshape(-1), d1, src_r, dst_r, c16)
    return _tc_post(S2, m2, zi)


def kernel(attr, d, edge_index, W0_0, W1_0, W2_0, Wa_0,
           W0_1, W1_1, W2_1, Wa_1):
    n, _ = attr.shape
    e = edge_index.shape[1]
    hh = W1_0.shape[0]
    src_r = edge_index[0].reshape(e // K, K)
    dst_r = edge_index[1].reshape(e // K, K)
    d1 = d[:, 0]
    sc_edge = _make_sc_edge(n, e, hh)
    h = _layer(attr, d1, src_r, dst_r, W0_0, W1_0, W2_0, Wa_0, sc_edge)
    h = _layer(h, d1, src_r, dst_r, W0_1, W1_1, W2_1, Wa_1, sc_edge)
    return h


# speedup vs baseline: 18.0656x; 18.0656x over previous
import functools

import jax
import jax.numpy as jnp
from jax import lax
from jax.experimental import pallas as pl
from jax.experimental.pallas import tpu as pltpu
from jax.experimental.pallas import tpu_sc as plsc

L = 16
K = 80
SUP = 25


def _tc_pre_body(h_ref, wz_ref, wzi_ref, v_ref, z_ref, zi_ref, pq_ref):
    hb = h_ref[...]
    z_ref[...] = jnp.dot(hb, wz_ref[...], preferred_element_type=jnp.float32)
    zi_ref[...] = jnp.dot(hb, wzi_ref[...], preferred_element_type=jnp.float32)
    pq_ref[...] = jnp.dot(hb, v_ref[...], preferred_element_type=jnp.float32)


def _tc_pre(h, Wz, Wzi, V, block=1000):
    n, dd = h.shape
    hh = Wz.shape[1]
    return pl.pallas_call(
        _tc_pre_body,
        grid=(n // block,),
        in_specs=[
            pl.BlockSpec((block, dd), lambda i: (i, 0)),
            pl.BlockSpec((dd, hh), lambda i: (0, 0)),
            pl.BlockSpec((dd, hh), lambda i: (0, 0)),
            pl.BlockSpec((dd, 2), lambda i: (0, 0)),
        ],
        out_specs=[
            pl.BlockSpec((block, hh), lambda i: (i, 0)),
            pl.BlockSpec((block, hh), lambda i: (i, 0)),
            pl.BlockSpec((block, 2), lambda i: (i, 0)),
        ],
        out_shape=[
            jax.ShapeDtypeStruct((n, hh), jnp.float32),
            jax.ShapeDtypeStruct((n, hh), jnp.float32),
            jax.ShapeDtypeStruct((n, 2), jnp.float32),
        ],
    )(h, Wz, Wzi, V)


def _tc_post_body(s_ref, den_ref, m_ref, zi_ref, o_ref):
    mv = m_ref[...]
    mm = jnp.max(mv)
    wv = jnp.exp(mv - mm)
    w0 = wv[0, 0]
    w1 = wv[1, 0]
    num = s_ref[0] * w0 + s_ref[1] * w1
    dall = den_ref[...]
    den = (w0 * jnp.sum(dall[:, :16], axis=1)
           + w1 * jnp.sum(dall[:, 16:], axis=1))
    den = den[:, None]
    zn = jnp.where(den > 0, num / den, 0.0)
    o_ref[...] = jnp.maximum(zi_ref[...] + zn, 0.0)


def _tc_post(S2, den32, m2, zi, block=1000):
    n, hh = zi.shape
    return pl.pallas_call(
        _tc_post_body,
        grid=(n // block,),
        in_specs=[
            pl.BlockSpec((2, block, hh), lambda i: (0, i, 0)),
            pl.BlockSpec((block, 32), lambda i: (i, 0)),
            pl.BlockSpec((2, L), lambda i: (0, 0)),
            pl.BlockSpec((block, hh), lambda i: (i, 0)),
        ],
        out_specs=pl.BlockSpec((block, hh), lambda i: (i, 0)),
        out_shape=jax.ShapeDtypeStruct((n, hh), jnp.float32),
    )(S2, den32, m2, zi)


def _make_sc_edge(n, e, hh):
    info = plsc.get_sparse_core_info()
    nc, ns = info.num_cores, info.num_subcores
    nw = nc * ns
    ep = e // nw
    c2 = ep // K
    nsup = c2 // SUP
    nch = n // K
    tch = (nch + ns - 1) // ns
    mesh = plsc.VectorSubcoreMesh(core_axis_name="c", subcore_axis_name="s")

    @functools.partial(
        pl.kernel,
        out_type=[
            jax.ShapeDtypeStruct((nc, n, hh), jnp.float32),
            jax.ShapeDtypeStruct((nw * n,), jnp.float32),
            jax.ShapeDtypeStruct((nc * L,), jnp.float32),
        ],
        mesh=mesh,
        compiler_params=pltpu.CompilerParams(needs_layout_passes=False),
        scratch_types=[
            pltpu.VMEM((SUP, K), jnp.int32),
            pltpu.VMEM((SUP, K), jnp.int32),
            pltpu.VMEM((SUP * K,), jnp.float32),
            pltpu.VMEM((2 * n,), jnp.float32),
            pltpu.VMEM((n,), jnp.float32),
            pltpu.VMEM((K, hh), jnp.float32),
            pltpu.VMEM((L,), jnp.float32),
            pltpu.VMEM_SHARED((n, hh), jnp.float32),
            pltpu.VMEM_SHARED((ns * L,), jnp.float32),
            pltpu.SemaphoreType.DMA,
        ],
    )
    def sc_edge(z_hbm, pq_hbm, d_hbm, src_hbm, dst_hbm, c_hbm,
                s_out, den_out, m_out,
                src_v, dst_v, d_v, pq_v, den_v, zrows_v, c_v,
                s_sh, max_sh, sem):
        cid = lax.axis_index("c")
        sid = lax.axis_index("s")
        wid = cid * ns + sid
        pltpu.sync_copy(pq_hbm, pq_v)
        pltpu.sync_copy(c_hbm, c_v)
        cvec = c_v[...]

        def stage_edges(ss):
            pltpu.sync_copy(src_hbm.at[wid, ss], src_v)
            pltpu.sync_copy(dst_hbm.at[wid, ss], dst_v)
            pltpu.sync_copy(d_hbm.at[pl.ds(wid * ep + ss * SUP * K, SUP * K)],
                            d_v)

        def escore(r, u):
            sv = src_v[r, pl.ds(u * L, L)]
            dv = dst_v[r, pl.ds(u * L, L)]
            dd = d_v[pl.ds(r * K + u * L, L)]
            pg = plsc.load_gather(pq_v, [sv * 2])
            qg = plsc.load_gather(pq_v, [dv * 2 + 1])
            a = pg + qg + cvec * dd
            return dv, jnp.maximum(a, 0.01 * a)

        def p1s(ss, mxs):
            stage_edges(ss)

            def p1r(r, mxr):
                for u in range(K // L):
                    _, ev = escore(r, u)
                    mxr = jnp.maximum(mxr, ev)
                return mxr
            return lax.fori_loop(0, SUP, p1r, mxs)
        mx = lax.fori_loop(0, nsup, p1s,
                           jnp.full((L,), -jnp.inf, jnp.float32))

        def dz(i, _):
            den_v[pl.ds(i * L, L)] = jnp.zeros((L,), jnp.float32)
            return 0
        lax.fori_loop(0, n // L, dz, 0)

        def zf(i, _):
            def zr(j, _):
                zrows_v[i, pl.ds(j * L, L)] = jnp.zeros((L,), jnp.float32)
                return 0
            lax.fori_loop(0, hh // L, zr, 0)
            return 0
        lax.fori_loop(0, K, zf, 0)

        def zout(t, _):
            ch = t * ns + sid

            @pl.when(ch < nch)
            def _():
                pltpu.sync_copy(zrows_v, s_sh.at[pl.ds(ch * K, K)])
            return 0
        lax.fori_loop(0, tch, zout, 0)

        zrows_v[0, pl.ds(0, L)] = mx
        pltpu.sync_copy(zrows_v.at[0, pl.ds(0, L)],
                        max_sh.at[pl.ds(sid * L, L)])
        plsc.subcore_barrier()

        def rmax(i, acc):
            pltpu.sync_copy(max_sh.at[pl.ds(i * L, L)],
                            zrows_v.at[1, pl.ds(0, L)])
            return jnp.maximum(acc, zrows_v[1, pl.ds(0, L)])
        mxv = lax.fori_loop(0, ns, rmax,
                            jnp.full((L,), -jnp.inf, jnp.float32))
        m = jnp.max(mxv)

        def p2s(ss, _):
            stage_edges(ss)

            def p2r(r, _):
                cp = pltpu.async_copy(z_hbm.at[src_v.at[r]], zrows_v, sem)
                cp.wait()
                for u in range(K // L):
                    dv, ev = escore(r, u)
                    ee16 = jnp.exp(ev - m)
                    plsc.addupdate_scatter(den_v, [dv], ee16)
                    for r16 in range(L):
                        row = u * L + r16
                        sv16 = jnp.full((L,), ee16[r16], jnp.float32)
                        for cc in range(hh // L):
                            zrows_v[row, pl.ds(cc * L, L)] = (
                                zrows_v[row, pl.ds(cc * L, L)] * sv16)
                pltpu.sync_copy(zrows_v, s_sh.at[dst_v.at[r]], add=True)
                return 0
            lax.fori_loop(0, SUP, p2r, 0)
            return 0
        lax.fori_loop(0, nsup, p2s, 0)
        plsc.subcore_barrier()

        def cout(t, _):
            ch = t * ns + sid

            @pl.when(ch < nch)
            def _():
                pltpu.sync_copy(s_sh.at[pl.ds(ch * K, K)],
                                s_out.at[cid, pl.ds(ch * K, K)])
            return 0
        lax.fori_loop(0, tch, cout, 0)

        pltpu.sync_copy(den_v, den_out.at[pl.ds(wid * n, n)])

        @pl.when(sid == 0)
        def _():
            zrows_v[0, pl.ds(0, L)] = jnp.full((L,), m, jnp.float32)
            pltpu.sync_copy(zrows_v.at[0, pl.ds(0, L)],
                            m_out.at[pl.ds(cid * L, L)])

    return sc_edge


def _layer(h, d1, src_r, dst_r, W0, W1, W2, Wa, sc_edge):
    hh = W1.shape[0]
    wa1 = Wa[0, :hh]
    wa2 = Wa[0, hh:2 * hh]
    c = W0[0, 0] * Wa[0, 2 * hh]
    Wz = W1.T
    Wzi = W2.T
    V = jnp.stack([W1.T @ wa1, W1.T @ wa2], axis=1)
    c16 = jnp.full((L,), c, jnp.float32)
    z, zi, pq = _tc_pre(h, Wz, Wzi, V)
    n = h.shape[0]
    S2, den, m2 = sc_edge(z, pq.reshape(-1), d1, src_r, dst_r, c16)
    return _tc_post(S2, den.reshape(32, n).T, m2.reshape(2, L), zi)


def kernel(attr, d, edge_index, W0_0, W1_0, W2_0, Wa_0,
           W0_1, W1_1, W2_1, Wa_1):
    n, _ = attr.shape
    e = edge_index.shape[1]
    hh = W1_0.shape[0]
    nw = 32
    nsup = e // (nw * K * SUP)
    src_r = edge_index[0].reshape(nw, nsup, SUP, K)
    dst_r = edge_index[1].reshape(nw, nsup, SUP, K)
    d1 = d[:, 0]
    sc_edge = _make_sc_edge(n, e, hh)
    h = _layer(attr, d1, src_r, dst_r, W0_0, W1_0, W2_0, Wa_0, sc_edge)
    h = _layer(h, d1, src_r, dst_r, W0_1, W1_1, W2_1, Wa_1, sc_edge)
    return h
